# baseline (speedup 1.0000x reference)
import functools

import jax
import jax.numpy as jnp
from jax import lax
from jax.experimental import pallas as pl
from jax.experimental.pallas import tpu as pltpu
from jax.experimental.pallas import tpu_sc as plsc

N = 10000
E = 320000
D = 128
R = 16

NC = 2
NS = 16
NW = NC * NS
K = 80
ZR = 80
NZCH = N // ZR

BE = 2560
MB = BE // 128

CHUNKS = ((0, 63), (63, 62))

PI = 3.14159265358979323846

_mesh = plsc.VectorSubcoreMesh(core_axis_name="c", subcore_axis_name="s")


def _make_sc_angle(ebase, eh):
    epw = eh // NW

    def body(coord_hbm, ei_hbm, m_hbm, coord_v, row_v, col_v, m_v, sem):
        c = lax.axis_index("c")
        s = lax.axis_index("s")
        wid = s * NC + c
        base = ebase + wid * epw
        pltpu.async_copy(coord_hbm, coord_v, sem).wait()
        pltpu.async_copy(ei_hbm.at[pl.ds(base, epw)], row_v, sem).wait()
        pltpu.async_copy(ei_hbm.at[pl.ds(E + base, epw)], col_v, sem).wait()

        def grp(g, carry):
            off = g * 16
            r16 = row_v[pl.ds(off, 16)] * 3
            c16 = col_v[pl.ds(off, 16)] * 3
            xr = plsc.load_gather(coord_v, [r16])
            yr = plsc.load_gather(coord_v, [r16 + 1])
            zr = plsc.load_gather(coord_v, [r16 + 2])
            xc = plsc.load_gather(coord_v, [c16])
            yc = plsc.load_gather(coord_v, [c16 + 1])
            zc = plsc.load_gather(coord_v, [c16 + 2])
            dx = xr - xc
            dy = yr - yc
            dz = zr - zc
            q = dx * dx + dy * dy + dz * dz
            m_v[pl.ds(off, 16)] = jnp.minimum(q * jnp.float32(1e24),
                                              jnp.float32(1.0))
            return carry

        if epw % 32 == 0:
            def grp2(g, carry):
                grp(2 * g, carry)
                grp(2 * g + 1, carry)
                return carry

            lax.fori_loop(0, epw // 32, grp2, 0)
        else:
            lax.fori_loop(0, epw // 16, grp, 0)
        pltpu.sync_copy(m_v, m_hbm.at[pl.ds(wid * epw, epw)])

    return functools.partial(
        pl.kernel,
        out_type=jax.ShapeDtypeStruct((eh,), jnp.float32),
        mesh=_mesh,
        compiler_params=pltpu.CompilerParams(needs_layout_passes=False),
        scratch_types=[
            pltpu.VMEM((3 * N,), jnp.float32),
            pltpu.VMEM((epw,), jnp.int32),
            pltpu.VMEM((epw,), jnp.int32),
            pltpu.VMEM((epw,), jnp.float32),
            pltpu.SemaphoreType.DMA,
        ],
    )(body)


_sc_angles = [_make_sc_angle(b * BE, n * BE) for b, n in CHUNKS]


def _make_sc_scatter(ebase, eh, chained):
    epw = eh // NW
    nch = epw // K

    def body(msg_hbm, ei_hbm, *rest):
        if chained:
            (prev_hbm, out_hbm, acc, idx0, idx1, msg0, msg1,
             sem_i0, sem_i1, sem_m0, sem_m1) = rest
        else:
            (out_hbm, acc, idx0, idx1, msg0, msg1, zbuf,
             sem_i0, sem_i1, sem_m0, sem_m1) = rest
        c = lax.axis_index("c")
        s = lax.axis_index("s")
        wid = s * NC + c
        mbase = wid * epw
        ibase = ebase + wid * epw

        if not chained:
            def zb(r, carry):
                for k8 in range(8):
                    zbuf[r, pl.ds(k8 * 16, 16)] = jnp.zeros((16,),
                                                            jnp.float32)
                return carry

            lax.fori_loop(0, ZR, zb, 0)

        def zchunk(t, carry):
            j = s + t * NS

            @pl.when(j < NZCH)
            def _():
                if chained:
                    pltpu.sync_copy(prev_hbm.at[pl.ds(c * N + j * ZR, ZR)],
                                    acc.at[pl.ds(j * ZR, ZR)])
                else:
                    pltpu.sync_copy(zbuf, acc.at[pl.ds(j * ZR, ZR)])

            return carry

        lax.fori_loop(0, (NZCH + NS - 1) // NS, zchunk, 0)
        plsc.subcore_barrier()

        def start(g, idx_b, msg_b, sem_i, sem_m):
            pltpu.async_copy(ei_hbm.at[pl.ds(ibase + g * K, K)], idx_b, sem_i)
            pltpu.async_copy(msg_hbm.at[pl.ds(mbase + g * K, K)], msg_b,
                             sem_m)

        def wait(idx_b, msg_b, sem_i, sem_m):
            pltpu.make_async_copy(ei_hbm.at[pl.ds(ibase, K)], idx_b,
                                  sem_i).wait()
            pltpu.make_async_copy(msg_hbm.at[pl.ds(mbase, K)], msg_b,
                                  sem_m).wait()

        start(0, idx0, msg0, sem_i0, sem_m0)

        def pipe(i, carry):
            g1 = i * 2 + 1

            @pl.when(g1 < nch)
            def _():
                start(g1, idx1, msg1, sem_i1, sem_m1)

            wait(idx0, msg0, sem_i0, sem_m0)
            pltpu.sync_copy(msg0, acc.at[idx0], add=True)

            @pl.when(g1 + 1 < nch)
            def _():
                start(g1 + 1, idx0, msg0, sem_i0, sem_m0)

            @pl.when(g1 < nch)
            def _():
                wait(idx1, msg1, sem_i1, sem_m1)
                pltpu.sync_copy(msg1, acc.at[idx1], add=True)

            return carry

        lax.fori_loop(0, (nch + 1) // 2, pipe, 0)
        plsc.subcore_barrier()

        def dchunk(t, carry):
            j = s + t * NS

            @pl.when(j < NZCH)
            def _():
                pltpu.sync_copy(acc.at[pl.ds(j * ZR, ZR)],
                                out_hbm.at[pl.ds(c * N + j * ZR, ZR)])

            return carry

        lax.fori_loop(0, (NZCH + NS - 1) // NS, dchunk, 0)

    scratch = [
        pltpu.VMEM_SHARED((N, D), jnp.float32),
        pltpu.VMEM((K,), jnp.int32),
        pltpu.VMEM((K,), jnp.int32),
        pltpu.VMEM((K, D), jnp.float32),
        pltpu.VMEM((K, D), jnp.float32),
    ]
    if not chained:
        scratch.append(pltpu.VMEM((ZR, D), jnp.float32))
    scratch += [pltpu.SemaphoreType.DMA] * 4
    return functools.partial(
        pl.kernel,
        out_type=jax.ShapeDtypeStruct((NC * N, D), jnp.float32),
        mesh=_mesh,
        scratch_types=scratch,
    )(body)


_sc_scatters = [_make_sc_scatter(b * BE, n * BE, i > 0)
                for i, (b, n) in enumerate(CHUNKS)]


def _acos_poly(x):
    p = jnp.float32(-0.0012624911)
    for a in (0.0066700901, -0.0170881256, 0.0308918810, -0.0501743046,
              0.0889789874, -0.2145988016, 1.5707963050):
        p = p * x + jnp.float32(a)
    return jnp.sqrt(jnp.maximum(jnp.float32(1.0) - x, jnp.float32(0.0))) * p


def _tc_edge_body(rbft_ref, m_ref, w1rt_ref, w1a_ref, b1_ref, w2t_ref, b2_ref,
                  out_ref):
    mrow = m_ref[...].reshape(MB, 128)
    ang = jnp.float32(PI) - _acos_poly(mrow)
    ang_rep = jnp.broadcast_to(ang[:, None, :],
                               (MB, 128, 128)).reshape(BE, 128)
    sub = lax.broadcasted_iota(jnp.int32, (BE, 128), 0)
    lane = lax.broadcasted_iota(jnp.int32, (BE, 128), 1)
    m1 = jnp.where(lax.rem(sub, 128) == lane, ang_rep, jnp.float32(0.0))
    wb = jnp.broadcast_to(w1a_ref[...], (128, D))
    z_ang = jnp.dot(m1, wb, preferred_element_type=jnp.float32)
    z = lax.dot_general(rbft_ref[...], w1rt_ref[...],
                        (((0,), (0,)), ((), ())),
                        preferred_element_type=jnp.float32)
    z = z + z_ang + b1_ref[...]
    h = z * (jnp.float32(1.0) / (jnp.float32(1.0) + jnp.exp(-z)))
    out_ref[...] = jnp.dot(h.astype(jnp.bfloat16), w2t_ref[...],
                           preferred_element_type=jnp.float32) + b2_ref[...]


def _make_tc_edge(ebase, eh):
    bb = ebase // BE
    nblk = eh // BE

    def call(rbft, m3, w1rt, w1a, b1, w2t, b2):
        return pl.pallas_call(
            _tc_edge_body,
            grid=(nblk,),
            in_specs=[
                pl.BlockSpec((R, BE), lambda i: (0, i + bb)),
                pl.BlockSpec((1, MB, 128), lambda i: (i, 0, 0)),
                pl.BlockSpec((R, D), lambda i: (0, 0)),
                pl.BlockSpec((1, D), lambda i: (0, 0)),
                pl.BlockSpec((1, D), lambda i: (0, 0)),
                pl.BlockSpec((D, D), lambda i: (0, 0)),
                pl.BlockSpec((1, D), lambda i: (0, 0)),
            ],
            out_specs=pl.BlockSpec((BE, D), lambda i: (i, 0)),
            out_shape=jax.ShapeDtypeStruct((eh, D), jnp.float32),
        )(rbft, m3, w1rt, w1a, b1, w2t, b2)

    return call


_tc_edges = [_make_tc_edge(b * BE, n * BE) for b, n in CHUNKS]


BN = 2000


def _tc_node_body(*refs):
    x_ref = refs[0]
    prefs = refs[1:-5]
    w3t_ref, b3_ref, w4t_ref, b4_ref, out_ref = refs[-5:]
    agg = prefs[0][...]
    for p in prefs[1:]:
        agg = agg + p[...]
    t = jnp.dot(agg, w3t_ref[...],
                preferred_element_type=jnp.float32) + b3_ref[...]
    u = t * (jnp.float32(1.0) / (jnp.float32(1.0) + jnp.exp(-t)))
    upd = jnp.dot(u, w4t_ref[...],
                  preferred_element_type=jnp.float32) + b4_ref[...]
    out_ref[...] = x_ref[...] + upd


def _tc_node(x, parts, w3t, b3, w4t, b4):
    nb = N // BN
    pspecs = []
    pargs = []
    for p in parts:
        pspecs.append(pl.BlockSpec((BN, D), lambda i: (i, 0)))
        pspecs.append(pl.BlockSpec((BN, D), lambda i, nb=nb: (i + nb, 0)))
        pargs.append(p)
        pargs.append(p)
    return pl.pallas_call(
        _tc_node_body,
        grid=(nb,),
        in_specs=[pl.BlockSpec((BN, D), lambda i: (i, 0))] + pspecs + [
            pl.BlockSpec((D, D), lambda i: (0, 0)),
            pl.BlockSpec((1, D), lambda i: (0, 0)),
            pl.BlockSpec((D, D), lambda i: (0, 0)),
            pl.BlockSpec((1, D), lambda i: (0, 0)),
        ],
        out_specs=pl.BlockSpec((BN, D), lambda i: (i, 0)),
        out_shape=jax.ShapeDtypeStruct((N, D), jnp.float32),
    )(x, *pargs, w3t, b3, w4t, b4)


@jax.jit
def kernel(x, edge_index, coord, rbf_feature, W1, b1, W2, b2, W3, b3, W4, b4):
    ei_flat = edge_index.astype(jnp.int32).reshape(2 * E)
    coord_flat = coord.astype(jnp.float32).reshape(3 * N)

    w1rt = W1[:, :R].T
    w1a = W1[:, R].reshape(1, D)
    b1r = b1.reshape(1, D)
    w2t = W2.T.astype(jnp.bfloat16)
    b2r = b2.reshape(1, D)
    w3t = W3.T
    b3r = b3.reshape(1, D)
    w4t = W4.T
    b4r = b4.reshape(1, D)
    rbft = rbf_feature.T

    parts = None
    for (b, n), ang_k, edge_k, scat_k in zip(CHUNKS, _sc_angles, _tc_edges,
                                             _sc_scatters):
        m = ang_k(coord_flat, ei_flat)
        msg = edge_k(rbft, m.reshape(n, MB, 128),
                     w1rt, w1a, b1r, w2t, b2r)
        if parts is None:
            parts = scat_k(msg, ei_flat)
        else:
            parts = scat_k(msg, ei_flat, parts)
    return _tc_node(x, [parts], w3t, b3r, w4t, b4r)


# speedup vs baseline: 1.0131x; 1.0131x over previous
import functools

import jax
import jax.numpy as jnp
from jax import lax
from jax.experimental import pallas as pl
from jax.experimental.pallas import tpu as pltpu
from jax.experimental.pallas import tpu_sc as plsc

N = 10000
E = 320000
D = 128
R = 16

NC = 2
NS = 16
NW = NC * NS
K = 80
ZR = 80
NZCH = N // ZR

BE = 2560
MB = BE // 128

CHUNKS = ((0, 63), (63, 62))

PI = 3.14159265358979323846

_mesh = plsc.VectorSubcoreMesh(core_axis_name="c", subcore_axis_name="s")


def _make_sc_angle(ebase, eh):
    epw = eh // NW

    def body(coord_hbm, ei_hbm, m_hbm, coord_v, row_v, col_v, m_v, sem):
        c = lax.axis_index("c")
        s = lax.axis_index("s")
        wid = s * NC + c
        base = ebase + wid * epw
        pltpu.async_copy(coord_hbm, coord_v, sem).wait()
        pltpu.async_copy(ei_hbm.at[pl.ds(base, epw)], row_v, sem).wait()
        pltpu.async_copy(ei_hbm.at[pl.ds(E + base, epw)], col_v, sem).wait()

        def grp(g, carry):
            off = g * 16
            r16 = row_v[pl.ds(off, 16)] * 3
            c16 = col_v[pl.ds(off, 16)] * 3
            xr = plsc.load_gather(coord_v, [r16])
            yr = plsc.load_gather(coord_v, [r16 + 1])
            zr = plsc.load_gather(coord_v, [r16 + 2])
            xc = plsc.load_gather(coord_v, [c16])
            yc = plsc.load_gather(coord_v, [c16 + 1])
            zc = plsc.load_gather(coord_v, [c16 + 2])
            dx = xr - xc
            dy = yr - yc
            dz = zr - zc
            q = dx * dx + dy * dy + dz * dz
            m_v[pl.ds(off, 16)] = jnp.minimum(q * jnp.float32(1e24),
                                              jnp.float32(1.0))
            return carry

        if epw % 32 == 0:
            def grp2(g, carry):
                grp(2 * g, carry)
                grp(2 * g + 1, carry)
                return carry

            lax.fori_loop(0, epw // 32, grp2, 0)
        else:
            lax.fori_loop(0, epw // 16, grp, 0)
        pltpu.sync_copy(m_v, m_hbm.at[pl.ds(wid * epw, epw)])

    return functools.partial(
        pl.kernel,
        out_type=jax.ShapeDtypeStruct((eh,), jnp.float32),
        mesh=_mesh,
        compiler_params=pltpu.CompilerParams(needs_layout_passes=False),
        scratch_types=[
            pltpu.VMEM((3 * N,), jnp.float32),
            pltpu.VMEM((epw,), jnp.int32),
            pltpu.VMEM((epw,), jnp.int32),
            pltpu.VMEM((epw,), jnp.float32),
            pltpu.SemaphoreType.DMA,
        ],
    )(body)


_sc_angles = [_make_sc_angle(b * BE, n * BE) for b, n in CHUNKS]


def _make_sc_scatter(ebase, eh, chained):
    epw = eh // NW
    nch = epw // K

    def body(msg_hbm, ei_hbm, *rest):
        if chained:
            (prev_hbm, out_hbm, acc, idx0, idx1, msg0, msg1,
             sem_i0, sem_i1, sem_m0, sem_m1) = rest
        else:
            (out_hbm, acc, idx0, idx1, msg0, msg1, zbuf,
             sem_i0, sem_i1, sem_m0, sem_m1) = rest
        c = lax.axis_index("c")
        s = lax.axis_index("s")
        wid = s * NC + c
        mbase = wid * epw
        ibase = ebase + wid * epw

        if not chained:
            def zb(r, carry):
                for k8 in range(8):
                    zbuf[r, pl.ds(k8 * 16, 16)] = jnp.zeros((16,),
                                                            jnp.float32)
                return carry

            lax.fori_loop(0, ZR, zb, 0)

        def zchunk(t, carry):
            j = s + t * NS

            @pl.when(j < NZCH)
            def _():
                if chained:
                    pltpu.sync_copy(prev_hbm.at[pl.ds(c * N + j * ZR, ZR)],
                                    acc.at[pl.ds(j * ZR, ZR)])
                else:
                    pltpu.sync_copy(zbuf, acc.at[pl.ds(j * ZR, ZR)])

            return carry

        lax.fori_loop(0, (NZCH + NS - 1) // NS, zchunk, 0)
        plsc.subcore_barrier()

        def start(g, idx_b, msg_b, sem_i, sem_m):
            pltpu.async_copy(ei_hbm.at[pl.ds(ibase + g * K, K)], idx_b, sem_i)
            pltpu.async_copy(msg_hbm.at[pl.ds(mbase + g * K, K)], msg_b,
                             sem_m)

        def wait(idx_b, msg_b, sem_i, sem_m):
            pltpu.make_async_copy(ei_hbm.at[pl.ds(ibase, K)], idx_b,
                                  sem_i).wait()
            pltpu.make_async_copy(msg_hbm.at[pl.ds(mbase, K)], msg_b,
                                  sem_m).wait()

        start(0, idx0, msg0, sem_i0, sem_m0)

        def pipe(i, carry):
            g1 = i * 2 + 1

            @pl.when(g1 < nch)
            def _():
                start(g1, idx1, msg1, sem_i1, sem_m1)

            wait(idx0, msg0, sem_i0, sem_m0)
            pltpu.sync_copy(msg0, acc.at[idx0], add=True)

            @pl.when(g1 + 1 < nch)
            def _():
                start(g1 + 1, idx0, msg0, sem_i0, sem_m0)

            @pl.when(g1 < nch)
            def _():
                wait(idx1, msg1, sem_i1, sem_m1)
                pltpu.sync_copy(msg1, acc.at[idx1], add=True)

            return carry

        lax.fori_loop(0, (nch + 1) // 2, pipe, 0)
        plsc.subcore_barrier()

        def dchunk(t, carry):
            j = s + t * NS

            @pl.when(j < NZCH)
            def _():
                pltpu.sync_copy(acc.at[pl.ds(j * ZR, ZR)],
                                out_hbm.at[pl.ds(c * N + j * ZR, ZR)])

            return carry

        lax.fori_loop(0, (NZCH + NS - 1) // NS, dchunk, 0)

    scratch = [
        pltpu.VMEM_SHARED((N, D), jnp.float32),
        pltpu.VMEM((K,), jnp.int32),
        pltpu.VMEM((K,), jnp.int32),
        pltpu.VMEM((K, D), jnp.float32),
        pltpu.VMEM((K, D), jnp.float32),
    ]
    if not chained:
        scratch.append(pltpu.VMEM((ZR, D), jnp.float32))
    scratch += [pltpu.SemaphoreType.DMA] * 4
    return functools.partial(
        pl.kernel,
        out_type=jax.ShapeDtypeStruct((NC * N, D), jnp.float32),
        mesh=_mesh,
        scratch_types=scratch,
    )(body)


_sc_scatters = [_make_sc_scatter(b * BE, n * BE, False) for b, n in CHUNKS]


def _acos_poly(x):
    p = jnp.float32(-0.0012624911)
    for a in (0.0066700901, -0.0170881256, 0.0308918810, -0.0501743046,
              0.0889789874, -0.2145988016, 1.5707963050):
        p = p * x + jnp.float32(a)
    return jnp.sqrt(jnp.maximum(jnp.float32(1.0) - x, jnp.float32(0.0))) * p


def _tc_edge_body(rbft_ref, m_ref, w1rt_ref, w1a_ref, b1_ref, w2t_ref, b2_ref,
                  out_ref):
    mrow = m_ref[...].reshape(MB, 128)
    ang = jnp.float32(PI) - _acos_poly(mrow)
    ang_rep = jnp.broadcast_to(ang[:, None, :],
                               (MB, 128, 128)).reshape(BE, 128)
    sub = lax.broadcasted_iota(jnp.int32, (BE, 128), 0)
    lane = lax.broadcasted_iota(jnp.int32, (BE, 128), 1)
    m1 = jnp.where(lax.rem(sub, 128) == lane, ang_rep, jnp.float32(0.0))
    wb = jnp.broadcast_to(w1a_ref[...], (128, D))
    z_ang = jnp.dot(m1, wb, preferred_element_type=jnp.float32)
    z = lax.dot_general(rbft_ref[...], w1rt_ref[...],
                        (((0,), (0,)), ((), ())),
                        preferred_element_type=jnp.float32)
    z = z + z_ang + b1_ref[...]
    h = z * (jnp.float32(1.0) / (jnp.float32(1.0) + jnp.exp(-z)))
    out_ref[...] = jnp.dot(h.astype(jnp.bfloat16), w2t_ref[...],
                           preferred_element_type=jnp.float32) + b2_ref[...]


def _make_tc_edge(ebase, eh):
    bb = ebase // BE
    nblk = eh // BE

    def call(rbft, m3, w1rt, w1a, b1, w2t, b2):
        return pl.pallas_call(
            _tc_edge_body,
            grid=(nblk,),
            in_specs=[
                pl.BlockSpec((R, BE), lambda i: (0, i + bb)),
                pl.BlockSpec((1, MB, 128), lambda i: (i, 0, 0)),
                pl.BlockSpec((R, D), lambda i: (0, 0)),
                pl.BlockSpec((1, D), lambda i: (0, 0)),
                pl.BlockSpec((1, D), lambda i: (0, 0)),
                pl.BlockSpec((D, D), lambda i: (0, 0)),
                pl.BlockSpec((1, D), lambda i: (0, 0)),
            ],
            out_specs=pl.BlockSpec((BE, D), lambda i: (i, 0)),
            out_shape=jax.ShapeDtypeStruct((eh, D), jnp.float32),
        )(rbft, m3, w1rt, w1a, b1, w2t, b2)

    return call


_tc_edges = [_make_tc_edge(b * BE, n * BE) for b, n in CHUNKS]


BN = 2000


def _tc_node_body(*refs):
    x_ref = refs[0]
    prefs = refs[1:-5]
    w3t_ref, b3_ref, w4t_ref, b4_ref, out_ref = refs[-5:]
    agg = prefs[0][...]
    for p in prefs[1:]:
        agg = agg + p[...]
    t = jnp.dot(agg, w3t_ref[...],
                preferred_element_type=jnp.float32) + b3_ref[...]
    u = t * (jnp.float32(1.0) / (jnp.float32(1.0) + jnp.exp(-t)))
    upd = jnp.dot(u, w4t_ref[...],
                  preferred_element_type=jnp.float32) + b4_ref[...]
    out_ref[...] = x_ref[...] + upd


def _tc_node(x, parts, w3t, b3, w4t, b4):
    nb = N // BN
    pspecs = []
    pargs = []
    for p in parts:
        pspecs.append(pl.BlockSpec((BN, D), lambda i: (i, 0)))
        pspecs.append(pl.BlockSpec((BN, D), lambda i, nb=nb: (i + nb, 0)))
        pargs.append(p)
        pargs.append(p)
    return pl.pallas_call(
        _tc_node_body,
        grid=(nb,),
        in_specs=[pl.BlockSpec((BN, D), lambda i: (i, 0))] + pspecs + [
            pl.BlockSpec((D, D), lambda i: (0, 0)),
            pl.BlockSpec((1, D), lambda i: (0, 0)),
            pl.BlockSpec((D, D), lambda i: (0, 0)),
            pl.BlockSpec((1, D), lambda i: (0, 0)),
        ],
        out_specs=pl.BlockSpec((BN, D), lambda i: (i, 0)),
        out_shape=jax.ShapeDtypeStruct((N, D), jnp.float32),
    )(x, *pargs, w3t, b3, w4t, b4)


@jax.jit
def kernel(x, edge_index, coord, rbf_feature, W1, b1, W2, b2, W3, b3, W4, b4):
    ei_flat = edge_index.astype(jnp.int32).reshape(2 * E)
    coord_flat = coord.astype(jnp.float32).reshape(3 * N)

    w1rt = W1[:, :R].T
    w1a = W1[:, R].reshape(1, D)
    b1r = b1.reshape(1, D)
    w2t = W2.T.astype(jnp.bfloat16)
    b2r = b2.reshape(1, D)
    w3t = W3.T
    b3r = b3.reshape(1, D)
    w4t = W4.T
    b4r = b4.reshape(1, D)
    rbft = rbf_feature.T

    parts = []
    for (b, n), ang_k, edge_k, scat_k in zip(CHUNKS, _sc_angles, _tc_edges,
                                             _sc_scatters):
        m = ang_k(coord_flat, ei_flat)
        msg = edge_k(rbft, m.reshape(n, MB, 128),
                     w1rt, w1a, b1r, w2t, b2r)
        parts.append(scat_k(msg, ei_flat))
    return _tc_node(x, parts, w3t, b3r, w4t, b4r)
